# baseline (speedup 1.0000x reference)
import functools

import jax
import jax.numpy as jnp
from jax import lax
from jax.experimental import pallas as pl
from jax.experimental.pallas import tpu as pltpu
from jax.experimental.pallas import tpu_sc as plsc

N = 50000
D = 64
R = 3
E = 65536

NC = 2
NS = 16
NW = NC * NS
EPW = E // NW
B = 128
NCH = EPW // B

S_AB = 2 * R
C_AB = S_AB * NCH
S_G = R
C_G = S_G * NCH

BLK = 2000


def _combo_body(nemb_ref, rmat_ref, out_ref):
    out_ref[:, 0:D] = jnp.dot(nemb_ref[...], rmat_ref[0],
                              preferred_element_type=jnp.float32)
    out_ref[:, D:2 * D] = nemb_ref[...]


def _compute_combo(node_emb, rel_mat):
    return pl.pallas_call(
        _combo_body,
        grid=(R, N // BLK),
        in_specs=[
            pl.BlockSpec((BLK, D), lambda r, n: (n, 0)),
            pl.BlockSpec((1, D, D), lambda r, n: (r, 0, 0)),
        ],
        out_specs=pl.BlockSpec((BLK, 2 * D), lambda r, n: (r * (N // BLK) + n, 0)),
        out_shape=jax.ShapeDtypeStruct((R * N, 2 * D), jnp.float32),
    )(node_emb, rel_mat)


TBLK = 512


def _gen_t_body(g_ref, out_ref):
    out_ref[...] = jnp.transpose(g_ref[0], (1, 0))


def _transpose_gen(gen_t):
    return pl.pallas_call(
        _gen_t_body,
        grid=(R, E // TBLK),
        in_specs=[pl.BlockSpec((1, D, TBLK), lambda r, i: (r, 0, i))],
        out_specs=pl.BlockSpec((TBLK, D), lambda r, i: (r * (E // TBLK) + i, 0)),
        out_shape=jax.ShapeDtypeStruct((R * E, D), jnp.float32),
    )(gen_t)


def _shuffle(a, idx):
    dnums = lax.GatherDimensionNumbers(
        offset_dims=(), collapsed_slice_dims=(0,), start_index_map=(0,))
    return lax.gather(a, idx[:, None], dnums, (1,),
                      mode=lax.GatherScatterMode.PROMISE_IN_BOUNDS)


def _sc_body(c_hbm, gen_hbm, src_ab, dst_ab, sn2, out_ab, out2,
             si_cat, di_cat, a_bufs, b_bufs, g_bufs, sc_buf, sem0, sem1):
    wid = lax.axis_index("s") * NC + lax.axis_index("c")
    lanes = lax.iota(jnp.int32, 16)
    zero16 = jnp.zeros((16,), jnp.float32)
    sems = (sem0, sem1)

    for s in range(S_AB):
        base = (s % R) * E + (s // R) * (R * E) + wid * EPW
        pltpu.sync_copy(src_ab.at[pl.ds(base, EPW)],
                        si_cat.at[pl.ds(s * EPW, EPW)])
        pltpu.sync_copy(dst_ab.at[pl.ds(base, EPW)],
                        di_cat.at[pl.ds(s * EPW, EPW)])

    def adj_ab(i, carry):
        roff = lax.rem(i // (EPW // 16), R) * N
        si_cat[pl.ds(i * 16, 16)] = si_cat[pl.ds(i * 16, 16)] + roff
        di_cat[pl.ds(i * 16, 16)] = di_cat[pl.ds(i * 16, 16)] + roff
        return carry

    lax.fori_loop(0, S_AB * EPW // 16, adj_ab, 0)

    def fire_ab(t, p):
        pltpu.async_copy(c_hbm.at[si_cat.at[pl.ds(t * B, B)]],
                         a_bufs.at[p], sems[p])
        pltpu.async_copy(c_hbm.at[di_cat.at[pl.ds(t * B, B)]],
                         b_bufs.at[p], sems[p])

    def drain_ab(p):
        pltpu.make_async_copy(c_hbm.at[si_cat.at[pl.ds(0, B)]],
                              a_bufs.at[p], sems[p]).wait()
        pltpu.make_async_copy(c_hbm.at[di_cat.at[pl.ds(0, B)]],
                              b_bufs.at[p], sems[p]).wait()

    fire_ab(0, 0)
    fire_ab(1, 1)

    def chunk_ab(t, carry):
        par = lax.rem(t, 2)

        @pl.when(par == 0)
        def _():
            drain_ab(0)

        @pl.when(par == 1)
        def _():
            drain_ab(1)

        coff = lax.rem(t, NCH) * B

        def grp(g, carry2):
            svec = zero16
            for j in range(16):
                e = g * 16 + j
                acc = (a_bufs[par, e, pl.ds(0, 16)]
                       * b_bufs[par, e, pl.ds(D, 16)])
                for k in range(1, 4):
                    acc = acc + (a_bufs[par, e, pl.ds(k * 16, 16)]
                                 * b_bufs[par, e, pl.ds(D + k * 16, 16)])
                for dist in (1, 2, 4, 8):
                    acc = acc + _shuffle(acc, lanes ^ dist)
                svec = svec + jnp.where(lanes == j, acc, zero16)
            sc_buf[pl.ds(coff + g * 16, 16)] = svec
            return carry2

        lax.fori_loop(0, B // 16, grp, 0)

        @pl.when(t + 2 < C_AB)
        def _():
            @pl.when(par == 0)
            def _():
                fire_ab(t + 2, 0)

            @pl.when(par == 1)
            def _():
                fire_ab(t + 2, 1)

        @pl.when(lax.rem(t, NCH) == NCH - 1)
        def _():
            s = t // NCH
            obase = (s % R) * E + (s // R) * (R * E) + wid * EPW
            pltpu.sync_copy(sc_buf, out_ab.at[pl.ds(obase, EPW)])

        return carry

    lax.fori_loop(0, C_AB, chunk_ab, 0)

    for s in range(S_G):
        base = s * E + wid * EPW
        pltpu.sync_copy(sn2.at[pl.ds(base, EPW)],
                        si_cat.at[pl.ds(s * EPW, EPW)])

    def adj_g(i, carry):
        roff = (i // (EPW // 16)) * N
        si_cat[pl.ds(i * 16, 16)] = si_cat[pl.ds(i * 16, 16)] + roff
        return carry

    lax.fori_loop(0, S_G * EPW // 16, adj_g, 0)

    def fire_g(t, p):
        pltpu.async_copy(c_hbm.at[si_cat.at[pl.ds(t * B, B)]],
                         a_bufs.at[p], sems[p])
        goff = pl.multiple_of((t // NCH) * E + wid * EPW + lax.rem(t, NCH) * B,
                              8)
        pltpu.async_copy(gen_hbm.at[pl.ds(goff, B), :], g_bufs.at[p], sems[p])

    def drain_g(p):
        pltpu.make_async_copy(c_hbm.at[si_cat.at[pl.ds(0, B)]],
                              a_bufs.at[p], sems[p]).wait()
        pltpu.make_async_copy(gen_hbm.at[pl.ds(0, B), :],
                              g_bufs.at[p], sems[p]).wait()

    fire_g(0, 0)
    fire_g(1, 1)

    def chunk_g(t, carry):
        par = lax.rem(t, 2)

        @pl.when(par == 0)
        def _():
            drain_g(0)

        @pl.when(par == 1)
        def _():
            drain_g(1)

        coff = lax.rem(t, NCH) * B

        def grp(g, carry2):
            svec = zero16
            for j in range(16):
                e = g * 16 + j
                acc = (a_bufs[par, e, pl.ds(0, 16)]
                       * g_bufs[par, e, pl.ds(0, 16)])
                for k in range(1, 4):
                    acc = acc + (a_bufs[par, e, pl.ds(k * 16, 16)]
                                 * g_bufs[par, e, pl.ds(k * 16, 16)])
                for dist in (1, 2, 4, 8):
                    acc = acc + _shuffle(acc, lanes ^ dist)
                svec = svec + jnp.where(lanes == j, acc, zero16)
            sc_buf[pl.ds(coff + g * 16, 16)] = svec
            return carry2

        lax.fori_loop(0, B // 16, grp, 0)

        @pl.when(t + 2 < C_G)
        def _():
            @pl.when(par == 0)
            def _():
                fire_g(t + 2, 0)

            @pl.when(par == 1)
            def _():
                fire_g(t + 2, 1)

        @pl.when(lax.rem(t, NCH) == NCH - 1)
        def _():
            obase = (t // NCH) * E + wid * EPW
            pltpu.sync_copy(sc_buf, out2.at[pl.ds(obase, EPW)])

        return carry

    lax.fori_loop(0, C_G, chunk_g, 0)


_sc_kernel = functools.partial(
    pl.kernel,
    out_type=(
        jax.ShapeDtypeStruct((2 * R * E,), jnp.float32),
        jax.ShapeDtypeStruct((R * E,), jnp.float32),
    ),
    mesh=plsc.VectorSubcoreMesh(core_axis_name="c", subcore_axis_name="s"),
    scratch_types=[
        pltpu.VMEM((S_AB * EPW,), jnp.int32),
        pltpu.VMEM((S_AB * EPW,), jnp.int32),
        pltpu.VMEM((2, B, 2 * D), jnp.float32),
        pltpu.VMEM((2, B, 2 * D), jnp.float32),
        pltpu.VMEM((2, B, D), jnp.float32),
        pltpu.VMEM((EPW,), jnp.float32),
        pltpu.SemaphoreType.DMA,
        pltpu.SemaphoreType.DMA,
    ],
)(_sc_body)


def kernel(gen_emb, node_emb, rel_mat, src_pos, dst_pos,
           src_neg1, dst_neg1, src_neg2, dst_neg2):
    combo = _compute_combo(node_emb, rel_mat)
    src_ab = jnp.concatenate([src_pos.reshape(-1), src_neg1.reshape(-1)])
    dst_ab = jnp.concatenate([dst_pos.reshape(-1), dst_neg1.reshape(-1)])
    gen_rows = _transpose_gen(jnp.transpose(gen_emb, (0, 2, 1)))
    out_ab, out2 = _sc_kernel(
        combo, gen_rows,
        src_ab, dst_ab, src_neg2.reshape(-1),
    )
    return (out_ab[:R * E], out_ab[R * E:], out2)


# speedup vs baseline: 1.5215x; 1.5215x over previous
import functools

import jax
import jax.numpy as jnp
from jax import lax
from jax.experimental import pallas as pl
from jax.experimental.pallas import tpu as pltpu
from jax.experimental.pallas import tpu_sc as plsc

N = 50000
D = 64
R = 3
E = 65536

NC = 2
NS = 16
NW = NC * NS
EPW = E // NW
B = 128
NCH = EPW // B

S_AB = 2 * R
C_AB = S_AB * NCH
S_G = R
C_G = S_G * NCH

BLK = 2000


def _combo_body(nemb_ref, rmat_ref, out_ref):
    out_ref[:, 0:D] = jnp.dot(nemb_ref[...], rmat_ref[0],
                              preferred_element_type=jnp.float32)
    out_ref[:, D:2 * D] = nemb_ref[...]


def _compute_combo(node_emb, rel_mat):
    return pl.pallas_call(
        _combo_body,
        grid=(R, N // BLK),
        in_specs=[
            pl.BlockSpec((BLK, D), lambda r, n: (n, 0)),
            pl.BlockSpec((1, D, D), lambda r, n: (r, 0, 0)),
        ],
        out_specs=pl.BlockSpec((BLK, 2 * D), lambda r, n: (r * (N // BLK) + n, 0)),
        out_shape=jax.ShapeDtypeStruct((R * N, 2 * D), jnp.float32),
    )(node_emb, rel_mat)


def _shuffle(a, idx):
    dnums = lax.GatherDimensionNumbers(
        offset_dims=(), collapsed_slice_dims=(0,), start_index_map=(0,))
    return lax.gather(a, idx[:, None], dnums, (1,),
                      mode=lax.GatherScatterMode.PROMISE_IN_BOUNDS)


def _sc_body(c_hbm, gen_hbm, src_ab, dst_ab, sn2, out_ab, out2,
             si_cat, di_cat, a_bufs, b_bufs, g_bufs, sc_buf, sem0, sem1):
    wid = lax.axis_index("s") * NC + lax.axis_index("c")
    lanes = lax.iota(jnp.int32, 16)
    zero16 = jnp.zeros((16,), jnp.float32)
    sems = (sem0, sem1)

    for s in range(S_AB):
        base = (s % R) * E + (s // R) * (R * E) + wid * EPW
        pltpu.sync_copy(src_ab.at[pl.ds(base, EPW)],
                        si_cat.at[pl.ds(s * EPW, EPW)])
        pltpu.sync_copy(dst_ab.at[pl.ds(base, EPW)],
                        di_cat.at[pl.ds(s * EPW, EPW)])

    def adj_ab(i, carry):
        roff = lax.rem(i // (EPW // 16), R) * N
        si_cat[pl.ds(i * 16, 16)] = si_cat[pl.ds(i * 16, 16)] + roff
        di_cat[pl.ds(i * 16, 16)] = di_cat[pl.ds(i * 16, 16)] + roff
        return carry

    lax.fori_loop(0, S_AB * EPW // 16, adj_ab, 0)

    def fire_ab(t, p):
        pltpu.async_copy(c_hbm.at[si_cat.at[pl.ds(t * B, B)]],
                         a_bufs.at[p], sems[p])
        pltpu.async_copy(c_hbm.at[di_cat.at[pl.ds(t * B, B)]],
                         b_bufs.at[p], sems[p])

    def drain_ab(p):
        pltpu.make_async_copy(c_hbm.at[si_cat.at[pl.ds(0, B)]],
                              a_bufs.at[p], sems[p]).wait()
        pltpu.make_async_copy(c_hbm.at[di_cat.at[pl.ds(0, B)]],
                              b_bufs.at[p], sems[p]).wait()

    fire_ab(0, 0)
    fire_ab(1, 1)

    def chunk_ab(t, carry):
        par = lax.rem(t, 2)

        @pl.when(par == 0)
        def _():
            drain_ab(0)

        @pl.when(par == 1)
        def _():
            drain_ab(1)

        coff = lax.rem(t, NCH) * B

        def grp(g, carry2):
            svec = zero16
            for j in range(16):
                e = g * 16 + j
                acc = (a_bufs[par, e, pl.ds(0, 16)]
                       * b_bufs[par, e, pl.ds(D, 16)])
                for k in range(1, 4):
                    acc = acc + (a_bufs[par, e, pl.ds(k * 16, 16)]
                                 * b_bufs[par, e, pl.ds(D + k * 16, 16)])
                for dist in (1, 2, 4, 8):
                    acc = acc + _shuffle(acc, lanes ^ dist)
                svec = svec + jnp.where(lanes == j, acc, zero16)
            sc_buf[pl.ds(coff + g * 16, 16)] = svec
            return carry2

        lax.fori_loop(0, B // 16, grp, 0)

        @pl.when(t + 2 < C_AB)
        def _():
            @pl.when(par == 0)
            def _():
                fire_ab(t + 2, 0)

            @pl.when(par == 1)
            def _():
                fire_ab(t + 2, 1)

        @pl.when(lax.rem(t, NCH) == NCH - 1)
        def _():
            s = t // NCH
            obase = (s % R) * E + (s // R) * (R * E) + wid * EPW
            pltpu.sync_copy(sc_buf, out_ab.at[pl.ds(obase, EPW)])

        return carry

    lax.fori_loop(0, C_AB, chunk_ab, 0)

    for s in range(S_G):
        base = s * E + wid * EPW
        pltpu.sync_copy(sn2.at[pl.ds(base, EPW)],
                        si_cat.at[pl.ds(s * EPW, EPW)])

    def adj_g(i, carry):
        roff = (i // (EPW // 16)) * N
        si_cat[pl.ds(i * 16, 16)] = si_cat[pl.ds(i * 16, 16)] + roff
        return carry

    lax.fori_loop(0, S_G * EPW // 16, adj_g, 0)

    def fire_g(t, p):
        pltpu.async_copy(c_hbm.at[si_cat.at[pl.ds(t * B, B)]],
                         a_bufs.at[p], sems[p])
        goff = pl.multiple_of((t // NCH) * E + wid * EPW + lax.rem(t, NCH) * B,
                              8)
        pltpu.async_copy(gen_hbm.at[pl.ds(goff, B), :], g_bufs.at[p], sems[p])

    def drain_g(p):
        pltpu.make_async_copy(c_hbm.at[si_cat.at[pl.ds(0, B)]],
                              a_bufs.at[p], sems[p]).wait()
        pltpu.make_async_copy(gen_hbm.at[pl.ds(0, B), :],
                              g_bufs.at[p], sems[p]).wait()

    fire_g(0, 0)
    fire_g(1, 1)

    def chunk_g(t, carry):
        par = lax.rem(t, 2)

        @pl.when(par == 0)
        def _():
            drain_g(0)

        @pl.when(par == 1)
        def _():
            drain_g(1)

        coff = lax.rem(t, NCH) * B

        def grp(g, carry2):
            svec = zero16
            for j in range(16):
                e = g * 16 + j
                acc = (a_bufs[par, e, pl.ds(0, 16)]
                       * g_bufs[par, e, pl.ds(0, 16)])
                for k in range(1, 4):
                    acc = acc + (a_bufs[par, e, pl.ds(k * 16, 16)]
                                 * g_bufs[par, e, pl.ds(k * 16, 16)])
                for dist in (1, 2, 4, 8):
                    acc = acc + _shuffle(acc, lanes ^ dist)
                svec = svec + jnp.where(lanes == j, acc, zero16)
            sc_buf[pl.ds(coff + g * 16, 16)] = svec
            return carry2

        lax.fori_loop(0, B // 16, grp, 0)

        @pl.when(t + 2 < C_G)
        def _():
            @pl.when(par == 0)
            def _():
                fire_g(t + 2, 0)

            @pl.when(par == 1)
            def _():
                fire_g(t + 2, 1)

        @pl.when(lax.rem(t, NCH) == NCH - 1)
        def _():
            obase = (t // NCH) * E + wid * EPW
            pltpu.sync_copy(sc_buf, out2.at[pl.ds(obase, EPW)])

        return carry

    lax.fori_loop(0, C_G, chunk_g, 0)


_sc_kernel = functools.partial(
    pl.kernel,
    out_type=(
        jax.ShapeDtypeStruct((2 * R * E,), jnp.float32),
        jax.ShapeDtypeStruct((R * E,), jnp.float32),
    ),
    mesh=plsc.VectorSubcoreMesh(core_axis_name="c", subcore_axis_name="s"),
    scratch_types=[
        pltpu.VMEM((S_AB * EPW,), jnp.int32),
        pltpu.VMEM((S_AB * EPW,), jnp.int32),
        pltpu.VMEM((2, B, 2 * D), jnp.float32),
        pltpu.VMEM((2, B, 2 * D), jnp.float32),
        pltpu.VMEM((2, B, D), jnp.float32),
        pltpu.VMEM((EPW,), jnp.float32),
        pltpu.SemaphoreType.DMA,
        pltpu.SemaphoreType.DMA,
    ],
)(_sc_body)


def kernel(gen_emb, node_emb, rel_mat, src_pos, dst_pos,
           src_neg1, dst_neg1, src_neg2, dst_neg2):
    combo = _compute_combo(node_emb, rel_mat)
    src_ab = jnp.concatenate([src_pos.reshape(-1), src_neg1.reshape(-1)])
    dst_ab = jnp.concatenate([dst_pos.reshape(-1), dst_neg1.reshape(-1)])
    out_ab, out2 = _sc_kernel(
        combo, gen_emb.reshape(R * E, D),
        src_ab, dst_ab, src_neg2.reshape(-1),
    )
    return (out_ab[:R * E], out_ab[R * E:], out2)
